# baseline (speedup 1.0000x reference)
import jax
import jax.numpy as jnp
from jax import lax
from jax.experimental import pallas as pl
from jax.experimental.pallas import tpu as pltpu
from jax.experimental.pallas import tpu_sc as plsc

N_NODES = 10000
D_FEAT = 256
UNITS = 256
N_EDGES = 160000

NCOL = 2
CH = UNITS // NCOL
NTILES = 16
CHUNK = 80
NCHUNKS = (N_EDGES // NTILES) // CHUNK
BLK = 16
NBLK = NCHUNKS // BLK
TAIL = NCHUNKS - NBLK * BLK
N_PAD = 10240
ROWS_PER_TILE = N_PAD // NTILES


def _matmul_body(x_ref, w_ref, o_ref):
    o_ref[0] = jnp.dot(x_ref[...], w_ref[...],
                       preferred_element_type=jnp.float32)


def _matmul_split(x, W):
    m_blk = 1000
    return pl.pallas_call(
        _matmul_body,
        grid=(N_NODES // m_blk, NCOL),
        in_specs=[
            pl.BlockSpec((m_blk, D_FEAT), lambda i, c: (i, 0)),
            pl.BlockSpec((D_FEAT, CH), lambda i, c: (0, c)),
        ],
        out_specs=pl.BlockSpec((1, m_blk, CH), lambda i, c: (c, i, 0)),
        out_shape=jax.ShapeDtypeStruct((NCOL, N_NODES, CH), jnp.float32),
    )(x, W)


def _sc_body(h_hbm, src_hbm, dst_hbm, ew_hbm, b_hbm, out_hbm,
             acc, srcblk, dstblk, ewblk, graw0, graw1, ssc0, ssc1, bvec,
             gsem0, gsem1, ssem0, ssem1):
    cc = lax.axis_index("c")
    ss = lax.axis_index("s")
    h_sub = h_hbm.at[cc]
    graw = (graw0, graw1)
    ssc = (ssc0, ssc1)
    gsem = (gsem0, gsem1)
    ssem = (ssem0, ssem1)

    pltpu.sync_copy(b_hbm.at[pl.ds(cc * CH, CH)], bvec)

    def fill_row(r, _):
        for g in range(CH // 16):
            sl = pl.ds(g * 16, 16)
            ssc0[r, sl] = bvec[sl]
        return 0

    lax.fori_loop(0, CHUNK, fill_row, 0)
    for j in range(ROWS_PER_TILE // CHUNK):
        pltpu.sync_copy(
            ssc0, acc.at[pl.ds(ss * ROWS_PER_TILE + j * CHUNK, CHUNK)])
    plsc.subcore_barrier()

    HC = CHUNK // 2

    def start_gather(l, b):
        pltpu.async_copy(h_sub.at[srcblk.at[l, pl.ds(0, HC)]],
                         graw[b].at[pl.ds(0, HC)], gsem[b])
        pltpu.async_copy(h_sub.at[srcblk.at[l, pl.ds(HC, HC)]],
                         graw[b].at[pl.ds(HC, HC)], gsem[b])

    def process(l, b, bchunks, lookahead=True):
        pltpu.make_async_copy(
            h_sub.at[srcblk.at[l, pl.ds(0, HC)]],
            graw[b].at[pl.ds(0, HC)], gsem[b]).wait()
        pltpu.make_async_copy(
            h_sub.at[srcblk.at[l, pl.ds(HC, HC)]],
            graw[b].at[pl.ds(HC, HC)], gsem[b]).wait()

        @pl.when(l >= 2)
        def _():
            pltpu.make_async_copy(
                ssc[b], acc.at[dstblk.at[0]], ssem[b]).wait()

        gsplat = jnp.full((16,), l, jnp.int32)

        @plsc.parallel_loop(0, CHUNK, unroll=8)
        def _(e):
            wv = plsc.load_gather(
                ewblk, [gsplat, jnp.full((16,), e, jnp.int32)])
            for c in range(CH // 16):
                sl = pl.ds(c * 16, 16)
                ssc[b][e, sl] = graw[b][e, sl] * wv

        if lookahead:
            @pl.when(l + 2 < bchunks)
            def _():
                start_gather(l + 2, b)
        pltpu.async_copy(ssc[b], acc.at[dstblk.at[l]], ssem[b], add=True)

    def load_meta(base, bchunks):
        sl_v = pl.ds(0, bchunks)
        sl_h = pl.ds(base, bchunks)
        pltpu.sync_copy(src_hbm.at[ss].at[sl_h], srcblk.at[sl_v])
        pltpu.sync_copy(dst_hbm.at[ss].at[sl_h], dstblk.at[sl_v])
        pltpu.sync_copy(ew_hbm.at[ss].at[sl_h], ewblk.at[sl_v])

    def drain_scatters():
        for b in range(2):
            pltpu.make_async_copy(
                ssc[b], acc.at[dstblk.at[0]], ssem[b]).wait()

    def blkbody(blk, _):
        load_meta(blk * BLK, BLK)
        start_gather(0, 0)
        start_gather(1, 1)

        def pair(k, _):
            process(2 * k, 0, BLK)
            process(2 * k + 1, 1, BLK)
            return 0

        lax.fori_loop(0, BLK // 2, pair, 0)
        drain_scatters()
        return 0

    lax.fori_loop(0, NBLK, blkbody, 0)

    load_meta(NBLK * BLK, TAIL)
    start_gather(0, 0)
    start_gather(1, 1)

    def tail_pair(k, _):
        process(2 * k, 0, TAIL)
        process(2 * k + 1, 1, TAIL)
        return 0

    lax.fori_loop(0, TAIL // 2, tail_pair, 0)
    process(TAIL - 1, 0, TAIL, lookahead=False)
    drain_scatters()
    plsc.subcore_barrier()

    r0 = ss * ROWS_PER_TILE
    csl = pl.ds(cc * CH, CH)

    @pl.when(r0 + ROWS_PER_TILE <= N_NODES)
    def _():
        pltpu.sync_copy(acc.at[pl.ds(r0, ROWS_PER_TILE)],
                        out_hbm.at[pl.ds(r0, ROWS_PER_TILE), csl])

    last = N_NODES - (NTILES - 1) * ROWS_PER_TILE

    @pl.when(r0 + ROWS_PER_TILE > N_NODES)
    def _():
        pltpu.sync_copy(acc.at[pl.ds(r0, last)],
                        out_hbm.at[pl.ds(r0, last), csl])


def kernel(x, edge_index, edge_weight, W, b):
    h_split = _matmul_split(x, W)
    src = edge_index[1].reshape(NTILES, NCHUNKS, CHUNK)
    dst = edge_index[0].reshape(NTILES, NCHUNKS, CHUNK)
    ew = edge_weight.reshape(NTILES, NCHUNKS, CHUNK)

    mesh = plsc.VectorSubcoreMesh(core_axis_name="c", subcore_axis_name="s")
    sc_fn = pl.kernel(
        _sc_body,
        out_type=jax.ShapeDtypeStruct((N_NODES, UNITS), jnp.float32),
        mesh=mesh,
        compiler_params=pltpu.CompilerParams(needs_layout_passes=False),
        scratch_types=[
            pltpu.VMEM_SHARED((N_PAD, CH), jnp.float32),
            pltpu.VMEM((BLK, CHUNK), jnp.int32),
            pltpu.VMEM((BLK, CHUNK), jnp.int32),
            pltpu.VMEM((BLK, CHUNK), jnp.float32),
            pltpu.VMEM((CHUNK, CH), jnp.float32),
            pltpu.VMEM((CHUNK, CH), jnp.float32),
            pltpu.VMEM((CHUNK, CH), jnp.float32),
            pltpu.VMEM((CHUNK, CH), jnp.float32),
            pltpu.VMEM((CH,), jnp.float32),
            pltpu.SemaphoreType.DMA,
            pltpu.SemaphoreType.DMA,
            pltpu.SemaphoreType.DMA,
            pltpu.SemaphoreType.DMA,
        ],
    )
    return sc_fn(h_split, src, dst, ew, b)


# speedup vs baseline: 1.0190x; 1.0190x over previous
import jax
import jax.numpy as jnp
from jax import lax
from jax.experimental import pallas as pl
from jax.experimental.pallas import tpu as pltpu
from jax.experimental.pallas import tpu_sc as plsc

N_NODES = 10000
D_FEAT = 256
UNITS = 256
N_EDGES = 160000

NCOL = 2
CH = UNITS // NCOL
NTILES = 16
CHUNK = 80
NCHUNKS = (N_EDGES // NTILES) // CHUNK
BLK = 16
NBLK = NCHUNKS // BLK
TAIL = NCHUNKS - NBLK * BLK
N_PAD = 10240
ROWS_PER_TILE = N_PAD // NTILES


def _matmul_body(x_ref, w_ref, o_ref):
    o_ref[0] = jnp.dot(x_ref[...], w_ref[...],
                       preferred_element_type=jnp.float32)


def _matmul_split(x, W):
    m_blk = 1000
    return pl.pallas_call(
        _matmul_body,
        grid=(N_NODES // m_blk, NCOL),
        in_specs=[
            pl.BlockSpec((m_blk, D_FEAT), lambda i, c: (i, 0)),
            pl.BlockSpec((D_FEAT, CH), lambda i, c: (0, c)),
        ],
        out_specs=pl.BlockSpec((1, m_blk, CH), lambda i, c: (c, i, 0)),
        out_shape=jax.ShapeDtypeStruct((NCOL, N_NODES, CH), jnp.float32),
    )(x, W)


def _sc_body(h_hbm, src_hbm, dst_hbm, ew_hbm, b_hbm, out_hbm,
             acc, srcblk, dstblk, ewblk, graw0, graw1, ssc0, ssc1, bvec,
             gsem0, gsem1, ssem0, ssem1):
    cc = lax.axis_index("c")
    ss = lax.axis_index("s")
    h_sub = h_hbm.at[cc]
    graw = (graw0, graw1)
    ssc = (ssc0, ssc1)
    gsem = (gsem0, gsem1)
    ssem = (ssem0, ssem1)

    pltpu.sync_copy(b_hbm.at[pl.ds(cc * CH, CH)], bvec)

    def fill_row(r, _):
        for g in range(CH // 16):
            sl = pl.ds(g * 16, 16)
            ssc0[r, sl] = bvec[sl]
        return 0

    lax.fori_loop(0, CHUNK, fill_row, 0)
    for j in range(ROWS_PER_TILE // CHUNK):
        pltpu.sync_copy(
            ssc0, acc.at[pl.ds(ss * ROWS_PER_TILE + j * CHUNK, CHUNK)])
    plsc.subcore_barrier()

    HC = CHUNK // 2

    def start_gather(l, b):
        pltpu.async_copy(h_sub.at[srcblk.at[l, pl.ds(0, HC)]],
                         graw[b].at[pl.ds(0, HC)], gsem[b])
        pltpu.async_copy(h_sub.at[srcblk.at[l, pl.ds(HC, HC)]],
                         graw[b].at[pl.ds(HC, HC)], gsem[b])

    def process(l, b, bchunks, lookahead=True):
        pltpu.make_async_copy(
            h_sub.at[srcblk.at[l, pl.ds(0, HC)]],
            graw[b].at[pl.ds(0, HC)], gsem[b]).wait()
        pltpu.make_async_copy(
            h_sub.at[srcblk.at[l, pl.ds(HC, HC)]],
            graw[b].at[pl.ds(HC, HC)], gsem[b]).wait()

        @pl.when(l >= 2)
        def _():
            pltpu.make_async_copy(
                ssc[b], acc.at[dstblk.at[0]], ssem[b]).wait()

        gsplat = jnp.full((16,), l, jnp.int32)

        @plsc.parallel_loop(0, CHUNK, unroll=16)
        def _(e):
            wv = plsc.load_gather(
                ewblk, [gsplat, jnp.full((16,), e, jnp.int32)])
            for c in range(CH // 16):
                sl = pl.ds(c * 16, 16)
                ssc[b][e, sl] = graw[b][e, sl] * wv

        if lookahead:
            @pl.when(l + 2 < bchunks)
            def _():
                start_gather(l + 2, b)
        pltpu.async_copy(ssc[b], acc.at[dstblk.at[l]], ssem[b], add=True)

    def load_meta(base, bchunks):
        sl_v = pl.ds(0, bchunks)
        sl_h = pl.ds(base, bchunks)
        pltpu.sync_copy(src_hbm.at[ss].at[sl_h], srcblk.at[sl_v])
        pltpu.sync_copy(dst_hbm.at[ss].at[sl_h], dstblk.at[sl_v])
        pltpu.sync_copy(ew_hbm.at[ss].at[sl_h], ewblk.at[sl_v])

    def drain_scatters():
        for b in range(2):
            pltpu.make_async_copy(
                ssc[b], acc.at[dstblk.at[0]], ssem[b]).wait()

    def blkbody(blk, _):
        load_meta(blk * BLK, BLK)
        start_gather(0, 0)
        start_gather(1, 1)

        def pair(k, _):
            process(2 * k, 0, BLK)
            process(2 * k + 1, 1, BLK)
            return 0

        lax.fori_loop(0, BLK // 2, pair, 0)
        drain_scatters()
        return 0

    lax.fori_loop(0, NBLK, blkbody, 0)

    load_meta(NBLK * BLK, TAIL)
    start_gather(0, 0)
    start_gather(1, 1)

    def tail_pair(k, _):
        process(2 * k, 0, TAIL)
        process(2 * k + 1, 1, TAIL)
        return 0

    lax.fori_loop(0, TAIL // 2, tail_pair, 0)
    process(TAIL - 1, 0, TAIL, lookahead=False)
    drain_scatters()
    plsc.subcore_barrier()

    r0 = ss * ROWS_PER_TILE
    csl = pl.ds(cc * CH, CH)

    @pl.when(r0 + ROWS_PER_TILE <= N_NODES)
    def _():
        pltpu.sync_copy(acc.at[pl.ds(r0, ROWS_PER_TILE)],
                        out_hbm.at[pl.ds(r0, ROWS_PER_TILE), csl])

    last = N_NODES - (NTILES - 1) * ROWS_PER_TILE

    @pl.when(r0 + ROWS_PER_TILE > N_NODES)
    def _():
        pltpu.sync_copy(acc.at[pl.ds(r0, last)],
                        out_hbm.at[pl.ds(r0, last), csl])


def kernel(x, edge_index, edge_weight, W, b):
    h_split = _matmul_split(x, W)
    src = edge_index[1].reshape(NTILES, NCHUNKS, CHUNK)
    dst = edge_index[0].reshape(NTILES, NCHUNKS, CHUNK)
    ew = edge_weight.reshape(NTILES, NCHUNKS, CHUNK)

    mesh = plsc.VectorSubcoreMesh(core_axis_name="c", subcore_axis_name="s")
    sc_fn = pl.kernel(
        _sc_body,
        out_type=jax.ShapeDtypeStruct((N_NODES, UNITS), jnp.float32),
        mesh=mesh,
        compiler_params=pltpu.CompilerParams(needs_layout_passes=False),
        scratch_types=[
            pltpu.VMEM_SHARED((N_PAD, CH), jnp.float32),
            pltpu.VMEM((BLK, CHUNK), jnp.int32),
            pltpu.VMEM((BLK, CHUNK), jnp.int32),
            pltpu.VMEM((BLK, CHUNK), jnp.float32),
            pltpu.VMEM((CHUNK, CH), jnp.float32),
            pltpu.VMEM((CHUNK, CH), jnp.float32),
            pltpu.VMEM((CHUNK, CH), jnp.float32),
            pltpu.VMEM((CHUNK, CH), jnp.float32),
            pltpu.VMEM((CH,), jnp.float32),
            pltpu.SemaphoreType.DMA,
            pltpu.SemaphoreType.DMA,
            pltpu.SemaphoreType.DMA,
            pltpu.SemaphoreType.DMA,
        ],
    )
    return sc_fn(h_split, src, dst, ew, b)


# speedup vs baseline: 1.0193x; 1.0003x over previous
import jax
import jax.numpy as jnp
from jax import lax
from jax.experimental import pallas as pl
from jax.experimental.pallas import tpu as pltpu
from jax.experimental.pallas import tpu_sc as plsc

N_NODES = 10000
D_FEAT = 256
UNITS = 256
N_EDGES = 160000

NCOL = 2
CH = UNITS // NCOL
NTILES = 16
CHUNK = 80
NCHUNKS = (N_EDGES // NTILES) // CHUNK
BLK = 16
NBLK = NCHUNKS // BLK
TAIL = NCHUNKS - NBLK * BLK
N_PAD = 10240
ROWS_PER_TILE = N_PAD // NTILES


def _matmul_body(x_ref, w_ref, o_ref):
    o_ref[0] = jnp.dot(x_ref[...], w_ref[...],
                       preferred_element_type=jnp.float32)


def _matmul_split(x, W):
    m_blk = 1000
    return pl.pallas_call(
        _matmul_body,
        grid=(N_NODES // m_blk, NCOL),
        in_specs=[
            pl.BlockSpec((m_blk, D_FEAT), lambda i, c: (i, 0)),
            pl.BlockSpec((D_FEAT, CH), lambda i, c: (0, c)),
        ],
        out_specs=pl.BlockSpec((1, m_blk, CH), lambda i, c: (c, i, 0)),
        out_shape=jax.ShapeDtypeStruct((NCOL, N_NODES, CH), jnp.float32),
    )(x, W)


def _sc_body(h_hbm, src_hbm, dst_hbm, ew_hbm, b_hbm, out_hbm,
             acc, srcblk, dstblk, ewblk, graw0, graw1, ssc0, ssc1, bvec,
             gsem0, gsem1, ssem0, ssem1):
    cc = lax.axis_index("c")
    ss = lax.axis_index("s")
    h_sub = h_hbm.at[cc]
    graw = (graw0, graw1)
    ssc = (ssc0, ssc1)
    gsem = (gsem0, gsem1)
    ssem = (ssem0, ssem1)

    pltpu.sync_copy(b_hbm.at[pl.ds(cc * CH, CH)], bvec)

    def fill_row(r, _):
        for g in range(CH // 16):
            sl = pl.ds(g * 16, 16)
            ssc0[r, sl] = bvec[sl]
        return 0

    lax.fori_loop(0, CHUNK, fill_row, 0)
    for j in range(ROWS_PER_TILE // CHUNK):
        pltpu.sync_copy(
            ssc0, acc.at[pl.ds(ss * ROWS_PER_TILE + j * CHUNK, CHUNK)])
    plsc.subcore_barrier()

    def start_gather(l, b):
        pltpu.async_copy(h_sub.at[srcblk.at[l]], graw[b], gsem[b])

    def process(l, b, bchunks, lookahead=True):
        pltpu.make_async_copy(
            h_sub.at[srcblk.at[l]], graw[b], gsem[b]).wait()

        @pl.when(l >= 2)
        def _():
            pltpu.make_async_copy(
                ssc[b], acc.at[dstblk.at[0]], ssem[b]).wait()

        gsplat = jnp.full((16,), l, jnp.int32)

        @plsc.parallel_loop(0, CHUNK, unroll=16)
        def _(e):
            wv = plsc.load_gather(
                ewblk, [gsplat, jnp.full((16,), e, jnp.int32)])
            for c in range(CH // 16):
                sl = pl.ds(c * 16, 16)
                ssc[b][e, sl] = graw[b][e, sl] * wv

        if lookahead:
            @pl.when(l + 2 < bchunks)
            def _():
                start_gather(l + 2, b)
        pltpu.async_copy(ssc[b], acc.at[dstblk.at[l]], ssem[b], add=True)

    def load_meta(base, bchunks):
        sl_v = pl.ds(0, bchunks)
        sl_h = pl.ds(base, bchunks)
        pltpu.sync_copy(src_hbm.at[ss].at[sl_h], srcblk.at[sl_v])
        pltpu.sync_copy(dst_hbm.at[ss].at[sl_h], dstblk.at[sl_v])
        pltpu.sync_copy(ew_hbm.at[ss].at[sl_h], ewblk.at[sl_v])

    def drain_scatters():
        for b in range(2):
            pltpu.make_async_copy(
                ssc[b], acc.at[dstblk.at[0]], ssem[b]).wait()

    def blkbody(blk, _):
        load_meta(blk * BLK, BLK)
        start_gather(0, 0)
        start_gather(1, 1)

        def pair(k, _):
            process(2 * k, 0, BLK)
            process(2 * k + 1, 1, BLK)
            return 0

        lax.fori_loop(0, BLK // 2, pair, 0)
        drain_scatters()
        return 0

    lax.fori_loop(0, NBLK, blkbody, 0)

    load_meta(NBLK * BLK, TAIL)
    start_gather(0, 0)
    start_gather(1, 1)

    def tail_pair(k, _):
        process(2 * k, 0, TAIL)
        process(2 * k + 1, 1, TAIL)
        return 0

    lax.fori_loop(0, TAIL // 2, tail_pair, 0)
    process(TAIL - 1, 0, TAIL, lookahead=False)
    drain_scatters()
    plsc.subcore_barrier()

    r0 = ss * ROWS_PER_TILE
    csl = pl.ds(cc * CH, CH)

    @pl.when(r0 + ROWS_PER_TILE <= N_NODES)
    def _():
        pltpu.sync_copy(acc.at[pl.ds(r0, ROWS_PER_TILE)],
                        out_hbm.at[pl.ds(r0, ROWS_PER_TILE), csl])

    last = N_NODES - (NTILES - 1) * ROWS_PER_TILE

    @pl.when(r0 + ROWS_PER_TILE > N_NODES)
    def _():
        pltpu.sync_copy(acc.at[pl.ds(r0, last)],
                        out_hbm.at[pl.ds(r0, last), csl])


def kernel(x, edge_index, edge_weight, W, b):
    h_split = _matmul_split(x, W)
    src = edge_index[1].reshape(NTILES, NCHUNKS, CHUNK)
    dst = edge_index[0].reshape(NTILES, NCHUNKS, CHUNK)
    ew = edge_weight.reshape(NTILES, NCHUNKS, CHUNK)

    mesh = plsc.VectorSubcoreMesh(core_axis_name="c", subcore_axis_name="s")
    sc_fn = pl.kernel(
        _sc_body,
        out_type=jax.ShapeDtypeStruct((N_NODES, UNITS), jnp.float32),
        mesh=mesh,
        compiler_params=pltpu.CompilerParams(needs_layout_passes=False),
        scratch_types=[
            pltpu.VMEM_SHARED((N_PAD, CH), jnp.float32),
            pltpu.VMEM((BLK, CHUNK), jnp.int32),
            pltpu.VMEM((BLK, CHUNK), jnp.int32),
            pltpu.VMEM((BLK, CHUNK), jnp.float32),
            pltpu.VMEM((CHUNK, CH), jnp.float32),
            pltpu.VMEM((CHUNK, CH), jnp.float32),
            pltpu.VMEM((CHUNK, CH), jnp.float32),
            pltpu.VMEM((CHUNK, CH), jnp.float32),
            pltpu.VMEM((CH,), jnp.float32),
            pltpu.SemaphoreType.DMA,
            pltpu.SemaphoreType.DMA,
            pltpu.SemaphoreType.DMA,
            pltpu.SemaphoreType.DMA,
        ],
    )
    return sc_fn(h_split, src, dst, ew, b)
